# Initial kernel scaffold; baseline (speedup 1.0000x reference)
import jax
import jax.numpy as jnp
from jax.experimental import pallas as pl


def kernel(x, edge_index, W1, b1, W2, b2, Wl, bl):
    raise NotImplementedError("write your pallas kernel here")


# speedup vs baseline: 13.9396x; 13.9396x over previous
import functools

import jax
import jax.numpy as jnp
from jax import lax
from jax.experimental import pallas as pl
from jax.experimental.pallas import tpu as pltpu
from jax.experimental.pallas import tpu_sc as plsc

N = 10000
NPAD = 10240
E = 320000
NC = 2
NS = 16
NW = NC * NS
EPW = E // NW
CHUNK = 80
NCHUNK = EPW // CHUNK
RPT = NPAD // NS
ZROWS = 128
DEGW = 16


def _sc_mesh():
    return plsc.VectorSubcoreMesh(core_axis_name="c", subcore_axis_name="s")


def _make_deg_kernel():
    @functools.partial(
        pl.kernel,
        out_type=jax.ShapeDtypeStruct((NC, NPAD, DEGW), jnp.float32),
        mesh=_sc_mesh(),
        scratch_types=[
            pltpu.VMEM((CHUNK,), jnp.int32),
            pltpu.VMEM((CHUNK, DEGW), jnp.float32),
            pltpu.VMEM((ZROWS, DEGW), jnp.float32),
            pltpu.VMEM_SHARED((NPAD, DEGW), jnp.float32),
        ],
    )
    def deg_kernel(dst_hbm, out_hbm, dst_v, ones_v, zbuf, acc):
        c = lax.axis_index("c")
        s = lax.axis_index("s")
        wid = c * NS + s

        def fill(i, _):
            zbuf[i, pl.ds(0, 16)] = jnp.zeros((16,), jnp.float32)
            return 0

        lax.fori_loop(0, ZROWS, fill, 0)

        def fill1(i, _):
            ones_v[i, pl.ds(0, 16)] = jnp.ones((16,), jnp.float32)
            return 0

        lax.fori_loop(0, CHUNK, fill1, 0)

        for j in range(RPT // ZROWS):
            pltpu.sync_copy(zbuf, acc.at[pl.ds(s * RPT + j * ZROWS, ZROWS)])
        plsc.subcore_barrier()

        base = wid * EPW

        def body(i, _):
            pltpu.sync_copy(dst_hbm.at[pl.ds(base + i * CHUNK, CHUNK)], dst_v)
            pltpu.sync_copy(ones_v, acc.at[dst_v], add=True)
            return 0

        lax.fori_loop(0, NCHUNK, body, 0)
        plsc.subcore_barrier()
        r0 = s * RPT
        pltpu.sync_copy(acc.at[pl.ds(r0, RPT)], out_hbm.at[c, pl.ds(r0, RPT)])

    return deg_kernel


def _make_edge_kernel(D):
    @functools.partial(
        pl.kernel,
        out_type=jax.ShapeDtypeStruct((NC, NPAD, D), jnp.float32),
        mesh=_sc_mesh(),
        scratch_types=[
            pltpu.VMEM((CHUNK,), jnp.int32),
            pltpu.VMEM((CHUNK,), jnp.int32),
            pltpu.VMEM((CHUNK, D), jnp.float32),
            pltpu.VMEM((ZROWS, D), jnp.float32),
            pltpu.VMEM_SHARED((NPAD, D), jnp.float32),
            pltpu.SemaphoreType.DMA,
        ],
        compiler_params=pltpu.CompilerParams(use_tc_tiling_on_sc=False),
    )
    def edge_kernel(src_hbm, dst_hbm, h_hbm, out_hbm,
                    src_v, dst_v, rows_v, zbuf, acc, sem):
        c = lax.axis_index("c")
        s = lax.axis_index("s")
        wid = c * NS + s
        lanes = D // 16

        def fill(i, _):
            zbuf[i // lanes, pl.ds((i % lanes) * 16, 16)] = (
                jnp.zeros((16,), jnp.float32))
            return 0

        lax.fori_loop(0, ZROWS * lanes, fill, 0)

        for j in range(RPT // ZROWS):
            pltpu.sync_copy(zbuf, acc.at[pl.ds(s * RPT + j * ZROWS, ZROWS)])
        plsc.subcore_barrier()

        base = wid * EPW

        def body(i, _):
            off = base + i * CHUNK
            pltpu.sync_copy(src_hbm.at[pl.ds(off, CHUNK)], src_v)
            pltpu.sync_copy(dst_hbm.at[pl.ds(off, CHUNK)], dst_v)
            pltpu.async_copy(h_hbm.at[src_v], rows_v, sem).wait()
            pltpu.sync_copy(rows_v, acc.at[dst_v], add=True)
            return 0

        lax.fori_loop(0, NCHUNK, body, 0)
        plsc.subcore_barrier()
        r0 = s * RPT
        pltpu.sync_copy(acc.at[pl.ds(r0, RPT)], out_hbm.at[c, pl.ds(r0, RPT)])

    return edge_kernel


_deg_kernel = _make_deg_kernel()
_edge_kernel_128 = _make_edge_kernel(128)
_edge_kernel_16 = _make_edge_kernel(16)

_MM_BLK = 1000


def _dis_block(da_ref, db_ref):
    deg = da_ref[:, 0:1] + db_ref[:, 0:1] + 1.0
    return lax.rsqrt(deg)


def _mm1(x, W1, degA, degB):
    def body(x_ref, w_ref, da_ref, db_ref, o_ref):
        dis = _dis_block(da_ref, db_ref)
        h = jnp.dot(x_ref[...], w_ref[...], preferred_element_type=jnp.float32)
        o_ref[...] = h * dis

    return pl.pallas_call(
        body,
        grid=(N // _MM_BLK,),
        in_specs=[
            pl.BlockSpec((_MM_BLK, 128), lambda i: (i, 0)),
            pl.BlockSpec((128, 128), lambda i: (0, 0)),
            pl.BlockSpec((_MM_BLK, DEGW), lambda i: (i, 0)),
            pl.BlockSpec((_MM_BLK, DEGW), lambda i: (i, 0)),
        ],
        out_specs=pl.BlockSpec((_MM_BLK, 128), lambda i: (i, 0)),
        out_shape=jax.ShapeDtypeStruct((N, 128), jnp.float32),
    )(x, W1, degA, degB)


def _mm2(aggA, aggB, h1s, degA, degB, b1, W2):
    def body(aa_ref, ab_ref, hs_ref, da_ref, db_ref, b_ref, w_ref, o_ref):
        dis = _dis_block(da_ref, db_ref)
        z = (aa_ref[...] + ab_ref[...] + hs_ref[...]) * dis + b_ref[...]
        z = jnp.maximum(z, 0.0)
        h2 = jnp.dot(z, w_ref[...], preferred_element_type=jnp.float32)
        o_ref[...] = h2 * dis

    return pl.pallas_call(
        body,
        grid=(N // _MM_BLK,),
        in_specs=[
            pl.BlockSpec((_MM_BLK, 128), lambda i: (i, 0)),
            pl.BlockSpec((_MM_BLK, 128), lambda i: (i, 0)),
            pl.BlockSpec((_MM_BLK, 128), lambda i: (i, 0)),
            pl.BlockSpec((_MM_BLK, DEGW), lambda i: (i, 0)),
            pl.BlockSpec((_MM_BLK, DEGW), lambda i: (i, 0)),
            pl.BlockSpec((1, 128), lambda i: (0, 0)),
            pl.BlockSpec((128, 16), lambda i: (0, 0)),
        ],
        out_specs=pl.BlockSpec((_MM_BLK, 16), lambda i: (i, 0)),
        out_shape=jax.ShapeDtypeStruct((N, 16), jnp.float32),
    )(aggA, aggB, h1s, degA, degB, b1, W2)


def _mm3(aggA, aggB, h2s, degA, degB, b2, Wl, bl):
    def body(aa_ref, ab_ref, hs_ref, da_ref, db_ref, b_ref, w_ref, bl_ref,
             o_ref):
        dis = _dis_block(da_ref, db_ref)
        z = (aa_ref[...] + ab_ref[...] + hs_ref[...]) * dis + b_ref[...]
        z = jnp.maximum(z, 0.0)
        o_ref[...] = (
            jnp.dot(z, w_ref[...], preferred_element_type=jnp.float32)
            + bl_ref[...])

    return pl.pallas_call(
        body,
        grid=(N // _MM_BLK,),
        in_specs=[
            pl.BlockSpec((_MM_BLK, 16), lambda i: (i, 0)),
            pl.BlockSpec((_MM_BLK, 16), lambda i: (i, 0)),
            pl.BlockSpec((_MM_BLK, 16), lambda i: (i, 0)),
            pl.BlockSpec((_MM_BLK, DEGW), lambda i: (i, 0)),
            pl.BlockSpec((_MM_BLK, DEGW), lambda i: (i, 0)),
            pl.BlockSpec((1, 16), lambda i: (0, 0)),
            pl.BlockSpec((16, 9), lambda i: (0, 0)),
            pl.BlockSpec((1, 9), lambda i: (0, 0)),
        ],
        out_specs=pl.BlockSpec((_MM_BLK, 9), lambda i: (i, 0)),
        out_shape=jax.ShapeDtypeStruct((N, 9), jnp.float32),
    )(aggA, aggB, h2s, degA, degB, b2, Wl, bl)


def kernel(x, edge_index, W1, b1, W2, b2, Wl, bl):
    ei = edge_index.astype(jnp.int32)
    src = ei[0]
    dst = ei[1]

    degp = _deg_kernel(dst)
    degA, degB = degp[0], degp[1]

    h1s = _mm1(x, W1, degA, degB)
    agg1 = _edge_kernel_128(src, dst, h1s)
    h2s = _mm2(agg1[0], agg1[1], h1s, degA, degB, b1.reshape(1, -1), W2)
    agg2 = _edge_kernel_16(src, dst, h2s)
    out = _mm3(agg2[0], agg2[1], h2s, degA, degB, b2.reshape(1, -1), Wl,
               bl.reshape(1, -1))
    return out
